# baseline (speedup 1.0000x reference)
import functools
import math

import jax
import jax.numpy as jnp
from jax import lax
from jax.experimental import pallas as pl
from jax.experimental.pallas import tpu as pltpu
from jax.experimental.pallas import tpu_sc as plsc

VOCAB_SIZE = 100000
D = 128
SCALE = math.sqrt(D)

NC = 2
NS = 16
NW = NC * NS
LANES = 16

CHUNK = 128
B_TOTAL = 4096 * 50
B_PER_W = B_TOTAL // NW
N_CHUNKS = B_PER_W // CHUNK


NBUF = 2


def _scale_chunk(src, dst):
    @pl.loop(0, CHUNK, unroll=4)
    def _row(r):
        for k in range(D // LANES):
            sl = pl.ds(k * LANES, LANES)
            dst[r, sl] = src[r, sl] * SCALE


def _body(tok_hbm, table_hbm, out_hbm, idx_v, in_v, out_v, gsem, wsem):
    wid = lax.axis_index("s") * NC + lax.axis_index("c")
    base = wid * B_PER_W

    pltpu.sync_copy(tok_hbm.at[wid], idx_v)

    def gather_start(c, b):
        pltpu.async_copy(table_hbm.at[idx_v.at[c]], in_v[b], gsem[b])

    def gather_wait(c, b):
        pltpu.make_async_copy(table_hbm.at[idx_v.at[c]], in_v[b],
                              gsem[b]).wait()

    def write_start(c, b):
        pltpu.async_copy(out_v[b], out_hbm.at[pl.ds(base + c * CHUNK, CHUNK)],
                         wsem[b])

    def write_wait(c, b):
        pltpu.make_async_copy(out_v[b],
                              out_hbm.at[pl.ds(base + c * CHUNK, CHUNK)],
                              wsem[b]).wait()

    for b in range(NBUF):
        gather_start(b, b)

    @pl.loop(0, N_CHUNKS, step=NBUF)
    def _grp(j):
        for b in range(NBUF):
            c = j + b
            gather_wait(c, b)

            @pl.when(c >= NBUF)
            def _():
                write_wait(c - NBUF, b)

            _scale_chunk(in_v[b], out_v[b])

            @pl.when(c + NBUF < N_CHUNKS)
            def _():
                gather_start(c + NBUF, b)

            write_start(c, b)

    for b in range(NBUF):
        write_wait(N_CHUNKS - NBUF + b, b)


@functools.partial(jax.jit, static_argnums=())
def _embed(tokens3d, table):
    mesh = plsc.VectorSubcoreMesh(
        core_axis_name="c", subcore_axis_name="s",
        num_cores=NC, num_subcores=NS,
    )
    kern = pl.kernel(
        _body,
        out_type=jax.ShapeDtypeStruct((B_TOTAL, D), jnp.float32),
        mesh=mesh,
        scratch_types=[
            pltpu.VMEM((N_CHUNKS, CHUNK), jnp.int32),
            [pltpu.VMEM((CHUNK, D), jnp.float32) for _ in range(NBUF)],
            [pltpu.VMEM((CHUNK, D), jnp.float32) for _ in range(NBUF)],
            [pltpu.SemaphoreType.DMA for _ in range(NBUF)],
            [pltpu.SemaphoreType.DMA for _ in range(NBUF)],
        ],
    )
    return kern(tokens3d, table)


def kernel(tokens, table):
    tok = tokens.astype(jnp.int32).reshape(NW, N_CHUNKS, CHUNK)
    out = _embed(tok, table)
    return out.reshape(tokens.shape[0], tokens.shape[1], D)


# speedup vs baseline: 1.5673x; 1.5673x over previous
import functools
import math

import jax
import jax.numpy as jnp
from jax import lax
from jax.experimental import pallas as pl
from jax.experimental.pallas import tpu as pltpu
from jax.experimental.pallas import tpu_sc as plsc

ROWS = 4096
SEQ = 50
D = 128
SCALE = math.sqrt(D)

NC = 2
NS = 16
NW = NC * NS
LANES = 16

R_PER_W = ROWS // NW


def _body(tok_hbm, table_hbm, out_hbm, idx_v, rows_v, sem):
    wid = lax.axis_index("s") * NC + lax.axis_index("c")
    base = wid * R_PER_W

    pltpu.sync_copy(tok_hbm.at[pl.ds(base, R_PER_W)], idx_v)

    @pl.loop(0, R_PER_W)
    def _row(j):
        pltpu.async_copy(table_hbm.at[idx_v.at[j]], rows_v, sem).wait()

        @pl.loop(0, SEQ, unroll=2)
        def _tok(r):
            for k in range(D // LANES):
                sl = pl.ds(k * LANES, LANES)
                rows_v[r, sl] = rows_v[r, sl] * SCALE

        pltpu.sync_copy(rows_v, out_hbm.at[base + j])


@jax.jit
def _embed(tokens, table):
    mesh = plsc.VectorSubcoreMesh(
        core_axis_name="c", subcore_axis_name="s",
        num_cores=NC, num_subcores=NS,
    )
    kern = pl.kernel(
        _body,
        out_type=jax.ShapeDtypeStruct((ROWS, SEQ, D), jnp.float32),
        mesh=mesh,
        scratch_types=[
            pltpu.VMEM((R_PER_W, SEQ), jnp.int32),
            pltpu.VMEM((SEQ, D), jnp.float32),
            pltpu.SemaphoreType.DMA,
        ],
    )
    return kern(tokens, table)


def kernel(tokens, table):
    return _embed(tokens.astype(jnp.int32), table)


# speedup vs baseline: 2.7702x; 1.7675x over previous
import functools
import math

import jax
import jax.numpy as jnp
from jax import lax
from jax.experimental import pallas as pl
from jax.experimental.pallas import tpu as pltpu
from jax.experimental.pallas import tpu_sc as plsc

ROWS = 4096
SEQ = 50
D = 128
SCALE = math.sqrt(D)

NC = 2
NS = 16
NW = NC * NS
LANES = 16

R_PER_W = ROWS // NW
RCHUNK = 4
N_CHUNKS = R_PER_W // RCHUNK
NBUF = 2


def _body(tok_hbm, table_hbm, out_hbm, idx_v, in_v, out_v, gsem, wsem):
    wid = lax.axis_index("s") * NC + lax.axis_index("c")
    base = wid * R_PER_W

    pltpu.sync_copy(tok_hbm.at[pl.ds(base, R_PER_W)], idx_v)

    def gather_start(c, b):
        for r in range(RCHUNK):
            pltpu.async_copy(table_hbm.at[idx_v.at[c * RCHUNK + r]],
                             in_v[b].at[r], gsem[b])

    def gather_wait(c, b):
        for r in range(RCHUNK):
            pltpu.make_async_copy(table_hbm.at[idx_v.at[c * RCHUNK + r]],
                                  in_v[b].at[r], gsem[b]).wait()

    def write_start(c, b):
        pltpu.async_copy(
            out_v[b], out_hbm.at[pl.ds(base + c * RCHUNK, RCHUNK)], wsem[b])

    def write_wait(c, b):
        pltpu.make_async_copy(
            out_v[b], out_hbm.at[pl.ds(base + c * RCHUNK, RCHUNK)],
            wsem[b]).wait()

    def scale(b):
        @pl.loop(0, SEQ, unroll=2)
        def _tok(t):
            for r in range(RCHUNK):
                for k in range(D // LANES):
                    sl = pl.ds(k * LANES, LANES)
                    out_v[b][r, t, sl] = in_v[b][r, t, sl] * SCALE

    for b in range(NBUF):
        gather_start(b, b)

    @pl.loop(0, N_CHUNKS, step=NBUF)
    def _grp(j):
        for b in range(NBUF):
            c = j + b
            gather_wait(c, b)

            @pl.when(c >= NBUF)
            def _():
                write_wait(c - NBUF, b)

            scale(b)

            @pl.when(c + NBUF < N_CHUNKS)
            def _():
                gather_start(c + NBUF, b)

            write_start(c, b)

    for b in range(NBUF):
        write_wait(N_CHUNKS - NBUF + b, b)


@jax.jit
def _embed(tokens, table):
    mesh = plsc.VectorSubcoreMesh(
        core_axis_name="c", subcore_axis_name="s",
        num_cores=NC, num_subcores=NS,
    )
    kern = pl.kernel(
        _body,
        out_type=jax.ShapeDtypeStruct((ROWS, SEQ, D), jnp.float32),
        mesh=mesh,
        scratch_types=[
            pltpu.VMEM((R_PER_W, SEQ), jnp.int32),
            [pltpu.VMEM((RCHUNK, SEQ, D), jnp.float32) for _ in range(NBUF)],
            [pltpu.VMEM((RCHUNK, SEQ, D), jnp.float32) for _ in range(NBUF)],
            [pltpu.SemaphoreType.DMA for _ in range(NBUF)],
            [pltpu.SemaphoreType.DMA for _ in range(NBUF)],
        ],
    )
    return kern(tokens, table)


def kernel(tokens, table):
    return _embed(tokens.astype(jnp.int32), table)
